# initial kernel scaffold (unmeasured)
import jax
import jax.numpy as jnp
from jax import lax
from jax.experimental import pallas as pl
from jax.experimental.pallas import tpu as pltpu

N_DEV = 8


def kernel(x, w_mat):
    m_total, k_shard = x.shape
    k_total, n = w_mat.shape
    m_per = m_total // N_DEV
    k_per = k_total // N_DEV
    assert k_shard == k_per

    def body(x_ref, w_ref, out_ref, xsend, xg, wbf, send_sems, recv_sems):
        my = lax.axis_index("i")

        for r in range(N_DEV):
            xsend[r] = x_ref[pl.ds(r * m_per, m_per), :].astype(jnp.bfloat16)

        barrier = pltpu.get_barrier_semaphore()
        for d in range(1, N_DEV):
            peer = lax.rem(my + d, N_DEV)
            pl.semaphore_signal(
                barrier, inc=1,
                device_id=(peer,), device_id_type=pl.DeviceIdType.MESH,
            )
        pl.semaphore_wait(barrier, N_DEV - 1)

        rdmas = []
        for d in range(1, N_DEV):
            peer = lax.rem(my + d, N_DEV)
            rdma = pltpu.make_async_remote_copy(
                src_ref=xsend.at[peer],
                dst_ref=xg.at[d],
                send_sem=send_sems.at[d],
                recv_sem=recv_sems.at[d],
                device_id=(peer,),
                device_id_type=pl.DeviceIdType.MESH,
            )
            rdma.start()
            rdmas.append(rdma)

        wbf[...] = w_ref[...].astype(jnp.bfloat16)

        out_ref[...] = jnp.dot(
            xsend[my], wbf[pl.ds(my * k_per, k_per), :],
            preferred_element_type=jnp.float32,
        )

        for d in range(1, N_DEV):
            rdmas[d - 1].wait_recv()
            s = lax.rem(my - d + N_DEV, N_DEV)
            out_ref[...] += jnp.dot(
                xg[d], wbf[pl.ds(s * k_per, k_per), :],
                preferred_element_type=jnp.float32,
            )

        for d in range(1, N_DEV):
            rdmas[d - 1].wait_send()

    return pl.pallas_call(
        body,
        out_shape=jax.ShapeDtypeStruct((m_per, n), jnp.float32),
        in_specs=[
            pl.BlockSpec(memory_space=pltpu.VMEM),
            pl.BlockSpec(memory_space=pltpu.VMEM),
        ],
        out_specs=pl.BlockSpec(memory_space=pltpu.VMEM),
        scratch_shapes=[
            pltpu.VMEM((N_DEV, m_per, k_per), jnp.bfloat16),
            pltpu.VMEM((N_DEV, m_per, k_per), jnp.bfloat16),
            pltpu.VMEM((k_total, n), jnp.bfloat16),
            pltpu.SemaphoreType.DMA((N_DEV,)),
            pltpu.SemaphoreType.DMA((N_DEV,)),
        ],
        compiler_params=pltpu.CompilerParams(collective_id=0),
    )(x, w_mat)


# baseline (device time: 49732 ns/iter reference)
import jax
import jax.numpy as jnp
from jax import lax
from jax.experimental import pallas as pl
from jax.experimental.pallas import tpu as pltpu

N_DEV = 8


def kernel(x, w_mat):
    m_total, k_shard = x.shape
    k_total, n = w_mat.shape
    m_per = m_total // N_DEV
    k_per = k_total // N_DEV
    assert k_shard == k_per

    def body(x_ref, w_ref, out_ref, xsend, xg, wtmp, w_sems,
             send_sems, recv_sems):
        my = lax.axis_index("i")

        for r in range(N_DEV):
            xsend[r] = x_ref[pl.ds(r * m_per, m_per), :].astype(jnp.bfloat16)

        barrier = pltpu.get_barrier_semaphore()
        for d in range(1, N_DEV):
            peer = lax.rem(my + d, N_DEV)
            pl.semaphore_signal(
                barrier, inc=1,
                device_id=(peer,), device_id_type=pl.DeviceIdType.MESH,
            )
        pl.semaphore_wait(barrier, N_DEV - 1)

        rdmas = []
        for d in range(1, N_DEV):
            peer = lax.rem(my + d, N_DEV)
            rdma = pltpu.make_async_remote_copy(
                src_ref=xsend.at[peer],
                dst_ref=xg.at[d],
                send_sem=send_sems.at[d],
                recv_sem=recv_sems.at[d],
                device_id=(peer,),
                device_id_type=pl.DeviceIdType.MESH,
            )
            rdma.start()
            rdmas.append(rdma)

        def stream_w(d, slot):
            s = lax.rem(my - d + N_DEV, N_DEV)
            cp = pltpu.make_async_copy(
                w_ref.at[pl.ds(s * k_per, k_per), :],
                wtmp.at[slot],
                w_sems.at[slot],
            )
            cp.start()
            return cp

        cps = {0: stream_w(0, 0), 1: stream_w(1, 1)}
        for d in range(N_DEV):
            slot = d % 2
            cps[d].wait()
            if d == 0:
                xblk = xsend[my]
            else:
                rdmas[d - 1].wait_recv()
                xblk = xg[d]
            contrib = jnp.dot(
                xblk, wtmp[slot].astype(jnp.bfloat16),
                preferred_element_type=jnp.float32,
            )
            if d == 0:
                out_ref[...] = contrib
            else:
                out_ref[...] += contrib
            if d + 2 < N_DEV:
                cps[d + 2] = stream_w(d + 2, slot)

        for d in range(1, N_DEV):
            rdmas[d - 1].wait_send()

    return pl.pallas_call(
        body,
        out_shape=jax.ShapeDtypeStruct((m_per, n), jnp.float32),
        in_specs=[
            pl.BlockSpec(memory_space=pltpu.VMEM),
            pl.BlockSpec(memory_space=pl.ANY),
        ],
        out_specs=pl.BlockSpec(memory_space=pltpu.VMEM),
        scratch_shapes=[
            pltpu.VMEM((N_DEV, m_per, k_per), jnp.bfloat16),
            pltpu.VMEM((N_DEV, m_per, k_per), jnp.bfloat16),
            pltpu.VMEM((2, k_per, n), jnp.float32),
            pltpu.SemaphoreType.DMA((2,)),
            pltpu.SemaphoreType.DMA((N_DEV,)),
            pltpu.SemaphoreType.DMA((N_DEV,)),
        ],
        compiler_params=pltpu.CompilerParams(
            collective_id=0,
            vmem_limit_bytes=100 * 1024 * 1024,
        ),
    )(x, w_mat)
